# baseline (speedup 1.0000x reference)
import jax
import jax.numpy as jnp
from jax.experimental import pallas as pl
from jax.experimental.pallas import tpu as pltpu

H, CH, W = 128, 64, 256
NE2 = 8
NE3 = 64
O1 = 8
O2 = 12
HID = 32
LPB = 2
LANES = 128


def _leaky(x):
    return jnp.where(x > 0, x, 0.01 * x)


def _argmax0(a, n):
    mx = jnp.max(a, axis=0)
    iota = jax.lax.broadcasted_iota(jnp.int32, a.shape, 0)
    cand = jnp.where(a == mx[None, :], iota, n)
    return jnp.min(cand, axis=0).astype(jnp.int32)


def _mm0(a, b):
    return jax.lax.dot_general(
        a, b, (((0,), (0,)), ((), ())), preferred_element_type=jnp.float32)


def _routed(slab, b, m, h, ne, d, co):
    G = LANES // co
    sub = d // G
    acc = _mm0(b, m)
    rows = jax.lax.broadcasted_iota(jnp.int32, (sub, d), 0)
    cols = jax.lax.broadcasted_iota(jnp.int32, (sub, d), 1)
    for g in range(G):
        P = (cols == G * rows + g).astype(jnp.float32)
        hs = jnp.dot(P, h, preferred_element_type=jnp.float32)
        hm = (m[:, None, :] * hs[None]).reshape(ne * sub, W)
        og = _mm0(slab, hm)
        acc = acc + og[co * g:co * (g + 1)]
    return acc


def _line_kernel(x_ref, w10, b10, w11, b11, w12, b12,
                 w20, b20, w21, b21, w22, b22,
                 w30, b30, w31, b31, w32, b32, out_ref):
    for j in range(LPB):
        _one_line(j, x_ref, w10, b10, w11, b11, w12, b12,
                  w20, b20, w21, b21, w22, b22,
                  w30, b30, w31, b31, w32, b32, out_ref)


def _one_line(j, x_ref, w10, b10, w11, b11, w12, b12,
              w20, b20, w21, b21, w22, b22,
              w30, b30, w31, b31, w32, b32, out_ref):
    X = x_ref[j]

    h = _leaky(jnp.dot(w10[j], X, preferred_element_type=jnp.float32) + b10[j])
    h = _leaky(jnp.dot(w11[j], h, preferred_element_type=jnp.float32) + b11[j])
    s1 = jnp.dot(w12[j], h, preferred_element_type=jnp.float32) + b12[j]
    inds1 = _argmax0(s1, O1)

    e2 = jax.lax.broadcasted_iota(jnp.int32, (NE2, W), 0)
    m2 = (e2 == inds1[None, :]).astype(jnp.float32)
    sl2 = pl.ds(j * NE2, NE2)
    h = _leaky(_routed(w20[j], b20[sl2], m2, X, NE2, CH, HID))
    h = _leaky(_routed(w21[j], b21[sl2], m2, h, NE2, HID, HID))
    s2 = _routed(w22[j], b22[sl2], m2, h, NE2, HID, 16)
    inds2 = _argmax0(s2, 16)

    inds12_raw = inds1 * NE2 + inds2 - 2
    inds12 = jnp.clip(inds12_raw, 0, NE3 - 1)

    e3 = jax.lax.broadcasted_iota(jnp.int32, (NE3, W), 0)
    m3 = (e3 == inds12[None, :]).astype(jnp.float32)
    sl3 = pl.ds(j * NE3, NE3)
    h = _leaky(_routed(w30[j], b30[sl3], m3, X, NE3, CH, HID))
    h = _leaky(_routed(w31[j], b31[sl3], m3, h, NE3, HID, HID))
    s3 = _routed(w32[j], b32[sl3], m3, h, NE3, HID, 16)
    inds3 = _argmax0(s3, 16)

    out_ref[j, 0] = jnp.clip(inds12_raw * NE2 + inds3 - 2, 0, NE3 * NE2 - 1)


def _slab(w, ne, d, co):
    return w.reshape(H, ne * d * co // LANES, LANES)


def _pad16(w, b):
    wp = jnp.pad(w, ((0, 0), (0, 0), (0, 4)))
    bp = jnp.pad(b, ((0, 0), (0, 4)), constant_values=-1e9)
    return wp, bp


def kernel(x_in, c1_w0, c1_b0, c1_w1, c1_b1, c1_w2, c1_b2,
           c2_w0, c2_b0, c2_w1, c2_b1, c2_w2, c2_b2,
           c3_w0, c3_b0, c3_w1, c3_b1, c3_w2, c3_b2):
    x3 = jnp.transpose(x_in[0], (1, 0, 2))
    c2_w2p, c2_b2p = _pad16(c2_w2, c2_b2)
    c3_w2p, c3_b2p = _pad16(c3_w2, c3_b2)

    def s3d(r):
        return pl.BlockSpec((LPB, r, LANES), lambda h: (h, 0, 0))

    def s1w(shape):
        return pl.BlockSpec((LPB,) + shape, lambda h: (h, 0, 0))

    def sb(ne, co):
        return pl.BlockSpec((LPB * ne, co), lambda h: (h, 0))

    in_specs = [
        pl.BlockSpec((LPB, CH, W), lambda h: (h, 0, 0)),
        s1w((HID, CH)), s1w((HID, 1)),
        s1w((HID, HID)), s1w((HID, 1)),
        s1w((O1, HID)), s1w((O1, 1)),
        s3d(NE2 * CH * HID // LANES), sb(NE2, HID),
        s3d(NE2 * HID * HID // LANES), sb(NE2, HID),
        s3d(NE2 * HID * 16 // LANES), sb(NE2, 16),
        s3d(NE3 * CH * HID // LANES), sb(NE3, HID),
        s3d(NE3 * HID * HID // LANES), sb(NE3, HID),
        s3d(NE3 * HID * 16 // LANES), sb(NE3, 16),
    ]

    args = [
        x3,
        c1_w0, c1_b0.reshape(H, HID, 1),
        c1_w1, c1_b1.reshape(H, HID, 1),
        c1_w2, c1_b2.reshape(H, O1, 1),
        _slab(c2_w0, NE2, CH, HID), c2_b0,
        _slab(c2_w1, NE2, HID, HID), c2_b1,
        _slab(c2_w2p, NE2, HID, 16), c2_b2p,
        _slab(c3_w0, NE3, CH, HID), c3_b0,
        _slab(c3_w1, NE3, HID, HID), c3_b1,
        _slab(c3_w2p, NE3, HID, 16), c3_b2p,
    ]

    out = pl.pallas_call(
        _line_kernel,
        grid=(H // LPB,),
        in_specs=in_specs,
        out_specs=pl.BlockSpec((LPB, 1, W), lambda h: (h, 0, 0)),
        out_shape=jax.ShapeDtypeStruct((H, 1, W), jnp.int32),
        compiler_params=pltpu.CompilerParams(
            dimension_semantics=("arbitrary",),
        ),
    )(*args)

    return out.reshape(1, 1, H, W)


# speedup vs baseline: 1.4297x; 1.4297x over previous
import jax
import jax.numpy as jnp
from jax.experimental import pallas as pl
from jax.experimental.pallas import tpu as pltpu

H, CH, W = 128, 64, 256
NE2 = 8
NE3 = 64
O1 = 8
O2 = 12
HID = 32


def _leaky(x):
    return jnp.where(x > 0, x, 0.01 * x)


def _argmax0(a, n):
    mx = jnp.max(a, axis=0)
    iota = jax.lax.broadcasted_iota(jnp.int32, a.shape, 0)
    cand = jnp.where(a == mx[None, :], iota, n)
    return jnp.min(cand, axis=0).astype(jnp.int32)


def _routed(w_ref, hm):
    return jax.lax.dot_general(
        w_ref[0], hm, (((1,), (0,)), ((), ())),
        preferred_element_type=jnp.float32)


def _line_kernel(x_ref,
                 w10, b10, w11, b11, w12, b12,
                 w20, w21, w22, w30, w31, w32,
                 out_ref):
    X = x_ref[0]

    h = _leaky(jnp.dot(w10[0], X, preferred_element_type=jnp.float32) + b10[0])
    h = _leaky(jnp.dot(w11[0], h, preferred_element_type=jnp.float32) + b11[0])
    s1 = jnp.dot(w12[0], h, preferred_element_type=jnp.float32) + b12[0]
    inds1 = _argmax0(s1, O1)

    e_iota2 = jax.lax.broadcasted_iota(jnp.int32, (NE2, 1, W), 0)
    m2 = (e_iota2 == inds1[None, None, :]).astype(jnp.float32)
    m2_2d = m2.reshape(NE2, W)

    hm = jnp.concatenate([(X[None] * m2).reshape(NE2 * CH, W), m2_2d], axis=0)
    h = _leaky(_routed(w20, hm))
    hm = jnp.concatenate([(h[None] * m2).reshape(NE2 * HID, W), m2_2d], axis=0)
    h = _leaky(_routed(w21, hm))
    hm = jnp.concatenate([(h[None] * m2).reshape(NE2 * HID, W), m2_2d], axis=0)
    s2 = _routed(w22, hm)

    inds2 = _argmax0(s2, O2)
    inds12_raw = inds1 * 8 + inds2 - 2
    inds12 = jnp.clip(inds12_raw, 0, NE3 - 1)

    e_iota3 = jax.lax.broadcasted_iota(jnp.int32, (NE3, 1, W), 0)
    m3 = (e_iota3 == inds12[None, None, :]).astype(jnp.float32)
    m3_2d = m3.reshape(NE3, W)

    hm = jnp.concatenate([(X[None] * m3).reshape(NE3 * CH, W), m3_2d], axis=0)
    h = _leaky(_routed(w30, hm))
    hm = jnp.concatenate([(h[None] * m3).reshape(NE3 * HID, W), m3_2d], axis=0)
    h = _leaky(_routed(w31, hm))
    hm = jnp.concatenate([(h[None] * m3).reshape(NE3 * HID, W), m3_2d], axis=0)
    s3 = _routed(w32, hm)

    inds3 = _argmax0(s3, O2)
    out_ref[0, 0] = jnp.clip(inds12_raw * 8 + inds3 - 2, 0, 511)


def _waug(w, b, ne, ci, co):
    wt = w.reshape(H, ne, ci, co).transpose(0, 3, 1, 2).reshape(H, co, ne * ci)
    bt = b.reshape(H, ne, co).transpose(0, 2, 1)
    return jnp.concatenate([wt, bt], axis=2)


def kernel(x_in, c1_w0, c1_b0, c1_w1, c1_b1, c1_w2, c1_b2,
           c2_w0, c2_b0, c2_w1, c2_b1, c2_w2, c2_b2,
           c3_w0, c3_b0, c3_w1, c3_b1, c3_w2, c3_b2):
    x_t = jnp.transpose(x_in[0], (1, 0, 2))

    def wspec(co, k):
        return pl.BlockSpec((1, co, k), lambda h: (h, 0, 0))

    K2a = NE2 * CH + NE2
    K2b = NE2 * HID + NE2
    K3a = NE3 * CH + NE3
    K3b = NE3 * HID + NE3

    in_specs = [
        pl.BlockSpec((1, CH, W), lambda h: (h, 0, 0)),
        pl.BlockSpec((1, HID, CH), lambda h: (h, 0, 0)),
        pl.BlockSpec((1, HID, 1), lambda h: (h, 0, 0)),
        pl.BlockSpec((1, HID, HID), lambda h: (h, 0, 0)),
        pl.BlockSpec((1, HID, 1), lambda h: (h, 0, 0)),
        pl.BlockSpec((1, O1, HID), lambda h: (h, 0, 0)),
        pl.BlockSpec((1, O1, 1), lambda h: (h, 0, 0)),
        wspec(HID, K2a), wspec(HID, K2b), wspec(O2, K2b),
        wspec(HID, K3a), wspec(HID, K3b), wspec(O2, K3b),
    ]

    args = [
        x_t,
        c1_w0, c1_b0.reshape(H, HID, 1),
        c1_w1, c1_b1.reshape(H, HID, 1),
        c1_w2, c1_b2.reshape(H, O1, 1),
        _waug(c2_w0, c2_b0, NE2, CH, HID),
        _waug(c2_w1, c2_b1, NE2, HID, HID),
        _waug(c2_w2, c2_b2, NE2, HID, O2),
        _waug(c3_w0, c3_b0, NE3, CH, HID),
        _waug(c3_w1, c3_b1, NE3, HID, HID),
        _waug(c3_w2, c3_b2, NE3, HID, O2),
    ]

    out = pl.pallas_call(
        _line_kernel,
        grid=(H,),
        in_specs=in_specs,
        out_specs=pl.BlockSpec((1, 1, W), lambda h: (h, 0, 0)),
        out_shape=jax.ShapeDtypeStruct((H, 1, W), jnp.int32),
        compiler_params=pltpu.CompilerParams(
            dimension_semantics=("arbitrary",),
        ),
    )(*args)

    return out.reshape(1, 1, H, W)
